# baseline (speedup 1.0000x reference)
import functools

import jax
import jax.numpy as jnp
from jax.experimental import pallas as pl
from jax.experimental.pallas import tpu as pltpu

B, T, D = 1, 2048, 768
FF = 3072
E = 8
K = 2

FFB = 768
NF = FF // FFB


def _router_kernel(x_ref, wg_ref, bg_ref, tw_ref, ti_ref):
    logits = jnp.dot(x_ref[...], wg_ref[...],
                     preferred_element_type=jnp.float32) + bg_ref[...]
    m = jnp.max(logits, axis=-1, keepdims=True)
    ex = jnp.exp(logits - m)
    probs = ex / jnp.sum(ex, axis=-1, keepdims=True)
    p1 = jnp.max(probs, axis=-1, keepdims=True)
    i1 = jnp.argmax(probs, axis=-1, keepdims=True)
    lane = jax.lax.broadcasted_iota(jnp.int32, probs.shape, 1)
    masked = jnp.where(lane == i1, -jnp.inf, probs)
    p2 = jnp.max(masked, axis=-1, keepdims=True)
    i2 = jnp.argmax(masked, axis=-1, keepdims=True)
    s = p1 + p2
    tw_ref[...] = jnp.concatenate([p1 / s, p2 / s], axis=1)
    ti_ref[...] = jnp.concatenate([i1, i2], axis=1).astype(jnp.int32)


def _moe_kernel(x_ref, w1_ref, b1_ref, w2_ref, b2_ref, ti_ref, tw_ref,
                out_ref):
    e = pl.program_id(0)
    f = pl.program_id(1)

    @pl.when((e == 0) & (f == 0))
    def _init():
        out_ref[...] = jnp.zeros_like(out_ref)

    h = jnp.dot(x_ref[...].astype(jnp.bfloat16), w1_ref[0].astype(jnp.bfloat16),
                preferred_element_type=jnp.float32) + b1_ref[0]
    h = jax.nn.gelu(h)
    p = jnp.dot(h.astype(jnp.bfloat16), w2_ref[0].astype(jnp.bfloat16),
                preferred_element_type=jnp.float32)

    ti = ti_ref[...]
    tw = tw_ref[...]
    gate = jnp.sum(jnp.where(ti == e, tw, 0.0), axis=1, keepdims=True)

    @pl.when(f == 0)
    def _bias2():
        out_ref[...] += gate * b2_ref[0]

    out_ref[...] += gate * p


def kernel(x, Wg, bg, W1, b1, W2, b2):
    xs = x.reshape(T, D)

    tw, ti = pl.pallas_call(
        _router_kernel,
        out_shape=(
            jax.ShapeDtypeStruct((T, K), jnp.float32),
            jax.ShapeDtypeStruct((T, K), jnp.int32),
        ),
    )(xs, Wg, bg.reshape(1, E))

    out = pl.pallas_call(
        _moe_kernel,
        grid=(E, NF),
        in_specs=[
            pl.BlockSpec((T, D), lambda e, f: (0, 0)),
            pl.BlockSpec((1, D, FFB), lambda e, f: (e, 0, f)),
            pl.BlockSpec((1, 1, FFB), lambda e, f: (e, 0, f)),
            pl.BlockSpec((1, FFB, D), lambda e, f: (e, f, 0)),
            pl.BlockSpec((1, 1, D), lambda e, f: (e, 0, 0)),
            pl.BlockSpec((T, K), lambda e, f: (0, 0)),
            pl.BlockSpec((T, K), lambda e, f: (0, 0)),
        ],
        out_specs=pl.BlockSpec((T, D), lambda e, f: (0, 0)),
        out_shape=jax.ShapeDtypeStruct((T, D), jnp.float32),
        compiler_params=pltpu.CompilerParams(
            dimension_semantics=("arbitrary", "arbitrary"),
        ),
    )(xs, W1, b1.reshape(E, 1, FF), W2, b2.reshape(E, 1, D), ti, tw)

    combined = out.reshape(B, T, D)
    aux_loss = jnp.zeros((), dtype=x.dtype)
    return combined, aux_loss, ti.reshape(B, T, K), tw.reshape(B, T, K)


# speedup vs baseline: 1.5443x; 1.5443x over previous
import functools

import jax
import jax.numpy as jnp
from jax import lax
from jax.experimental import pallas as pl
from jax.experimental.pallas import tpu as pltpu
from jax.experimental.pallas import tpu_sc as plsc

B, T, D = 1, 2048, 768
FF = 3072
E = 8
K = 2

RB = 512
NB = (T * K) // RB + E - 1
NROWS = NB * RB
FFB = 1536
NF = FF // FFB

NC, NS = 2, 16
NW = NC * NS
BPW = T // NW


def _plan_kernel(x_ref, wg_ref, bg_ref,
                 tw_ref, ti_ref, posa_ref, posb_ref, be_ref, tot_ref):
    logits = jnp.dot(x_ref[...], wg_ref[...],
                     preferred_element_type=jnp.float32) + bg_ref[...]
    m = jnp.max(logits, axis=-1, keepdims=True)
    ex = jnp.exp(logits - m)
    probs = ex / jnp.sum(ex, axis=-1, keepdims=True)
    p1 = jnp.max(probs, axis=-1, keepdims=True)
    i1 = jnp.argmax(probs, axis=-1, keepdims=True)
    lane = jax.lax.broadcasted_iota(jnp.int32, probs.shape, 1)
    masked = jnp.where(lane == i1, -jnp.inf, probs)
    p2 = jnp.max(masked, axis=-1, keepdims=True)
    i2 = jnp.argmax(masked, axis=-1, keepdims=True)
    s = p1 + p2
    tw_ref[...] = jnp.concatenate([p1 / s, p2 / s], axis=1)
    ti_ref[...] = jnp.concatenate([i1, i2], axis=1).astype(jnp.int32)

    a1 = lane == i1
    a2 = lane == i2
    a = (a1 | a2).astype(jnp.int32)
    CH = 256
    af = a.astype(jnp.float32)
    tri = (jax.lax.broadcasted_iota(jnp.int32, (CH, CH), 0) >
           jax.lax.broadcasted_iota(jnp.int32, (CH, CH), 1)
           ).astype(jnp.float32)
    carry = jnp.zeros((1, E), jnp.float32)
    excl_parts = []
    for c in range(T // CH):
        ac = af[c * CH:(c + 1) * CH, :]
        excl_parts.append(
            jnp.dot(tri, ac, preferred_element_type=jnp.float32) + carry)
        carry = carry + jnp.sum(ac, axis=0, keepdims=True)
    excl = jnp.concatenate(excl_parts, axis=0).astype(jnp.int32)
    counts = carry.astype(jnp.int32)
    pc = ((counts + RB - 1) // RB) * RB
    parts = [jnp.zeros((1, 1), jnp.int32)]
    run = pc[:, 0:1]
    for e in range(1, E):
        parts.append(run)
        run = run + pc[:, e:e + 1]
    gs = jnp.concatenate(parts, axis=1)
    tot_ref[...] = run

    gsb = jnp.broadcast_to(gs, (T, E))
    posa = jnp.sum(jnp.where(a1, gsb + excl, 0), axis=1, keepdims=True)
    posb = jnp.sum(jnp.where(a2, gsb + excl, 0), axis=1, keepdims=True)
    posa_ref[...] = posa
    posb_ref[...] = posb

    lane128 = jax.lax.broadcasted_iota(jnp.int32, (1, 128), 1)
    blkstart = lane128 * RB
    acc = jnp.zeros((1, 128), jnp.int32)
    for e in range(E):
        acc += jnp.where(blkstart >= gs[:, e:e + 1], 1, 0)
    be_ref[...] = jnp.clip(acc - 1, 0, E - 1)


def _gmm_kernel(be_ref, tot_ref, xs_ref, w1_ref, b1_ref, w2_ref, b2_ref,
                ys_ref):
    i = pl.program_id(0)
    f = pl.program_id(1)

    @pl.when(i * RB < tot_ref[0])
    def _():
        h = jnp.dot(xs_ref[...], w1_ref[0],
                    preferred_element_type=jnp.float32) + b1_ref[0]
        h = jax.nn.gelu(h)
        p = jnp.dot(h, w2_ref[0], preferred_element_type=jnp.float32)

        @pl.when(f == 0)
        def _():
            ys_ref[...] = jnp.broadcast_to(b2_ref[0], ys_ref.shape)

        ys_ref[...] += p


def _last(tot_ref):
    return tot_ref[0] // RB - 1


def _w1_map(i, f, be_ref, tot_ref):
    ie = jnp.minimum(i, _last(tot_ref))
    fe = jnp.where(i <= _last(tot_ref), f, NF - 1)
    return (be_ref[ie], 0, fe)


def _b1_map(i, f, be_ref, tot_ref):
    ie = jnp.minimum(i, _last(tot_ref))
    fe = jnp.where(i <= _last(tot_ref), f, NF - 1)
    return (be_ref[ie], 0, fe)


def _w2_map(i, f, be_ref, tot_ref):
    ie = jnp.minimum(i, _last(tot_ref))
    fe = jnp.where(i <= _last(tot_ref), f, NF - 1)
    return (be_ref[ie], fe, 0)


def _b2_map(i, f, be_ref, tot_ref):
    ie = jnp.minimum(i, _last(tot_ref))
    return (be_ref[ie], 0, 0)


def _combine_kernel(ra_ref, rb_ref, tw_ref, out_ref):
    tw = tw_ref[...]
    out_ref[...] = ra_ref[...] * tw[:, 0:1] + rb_ref[...] * tw[:, 1:2]


def _sc_scatter_body(x_hbm, posa_hbm, posb_hbm, xs_hbm, idx_v, rows_v, sem):
    wid = lax.axis_index("s") * NC + lax.axis_index("c")
    base = wid * BPW
    pltpu.sync_copy(x_hbm.at[pl.ds(base, BPW)], rows_v)
    pltpu.sync_copy(posa_hbm.at[pl.ds(base, BPW)], idx_v)
    pltpu.async_copy(rows_v, xs_hbm.at[idx_v], sem).wait()
    pltpu.sync_copy(posb_hbm.at[pl.ds(base, BPW)], idx_v)
    pltpu.async_copy(rows_v, xs_hbm.at[idx_v], sem).wait()


def _sc_gather_body(ys_hbm, posa_hbm, posb_hbm, ra_hbm, rb_hbm, idx_v,
                    rows_v, sem):
    wid = lax.axis_index("s") * NC + lax.axis_index("c")
    base = wid * BPW
    pltpu.sync_copy(posa_hbm.at[pl.ds(base, BPW)], idx_v)
    pltpu.async_copy(ys_hbm.at[idx_v], rows_v, sem).wait()
    pltpu.sync_copy(rows_v, ra_hbm.at[pl.ds(base, BPW)])
    pltpu.sync_copy(posb_hbm.at[pl.ds(base, BPW)], idx_v)
    pltpu.async_copy(ys_hbm.at[idx_v], rows_v, sem).wait()
    pltpu.sync_copy(rows_v, rb_hbm.at[pl.ds(base, BPW)])


@functools.cache
def _get_sc_kernels():
    mesh = plsc.VectorSubcoreMesh(core_axis_name="c", subcore_axis_name="s",
                                  num_cores=NC, num_subcores=NS)
    scratch = [
        pltpu.VMEM((BPW,), jnp.int32),
        pltpu.VMEM((BPW, D), jnp.float32),
        pltpu.SemaphoreType.DMA,
    ]
    scatter = pl.kernel(
        _sc_scatter_body,
        out_type=jax.ShapeDtypeStruct((NROWS, D), jnp.float32),
        mesh=mesh, scratch_types=scratch)
    gather = pl.kernel(
        _sc_gather_body,
        out_type=(jax.ShapeDtypeStruct((T, D), jnp.float32),
                  jax.ShapeDtypeStruct((T, D), jnp.float32)),
        mesh=mesh, scratch_types=scratch)
    return scatter, gather


def kernel(x, Wg, bg, W1, b1, W2, b2):
    xs = x.reshape(T, D)

    tw, ti, posa, posb, be, tot = pl.pallas_call(
        _plan_kernel,
        out_shape=(
            jax.ShapeDtypeStruct((T, K), jnp.float32),
            jax.ShapeDtypeStruct((T, K), jnp.int32),
            jax.ShapeDtypeStruct((T, 1), jnp.int32),
            jax.ShapeDtypeStruct((T, 1), jnp.int32),
            jax.ShapeDtypeStruct((1, 128), jnp.int32),
            jax.ShapeDtypeStruct((1, 1), jnp.int32),
        ),
    )(xs, Wg, bg.reshape(1, E))

    posa1 = posa.reshape(T)
    posb1 = posb.reshape(T)

    sc_scatter, sc_gather = _get_sc_kernels()
    xsorted = sc_scatter(xs, posa1, posb1)

    ys = pl.pallas_call(
        _gmm_kernel,
        grid_spec=pltpu.PrefetchScalarGridSpec(
            num_scalar_prefetch=2,
            grid=(NB, NF),
            in_specs=[
                pl.BlockSpec((RB, D), lambda i, f, be, tot: (i, 0)),
                pl.BlockSpec((1, D, FFB), _w1_map),
                pl.BlockSpec((1, 1, FFB), _b1_map),
                pl.BlockSpec((1, FFB, D), _w2_map),
                pl.BlockSpec((1, 1, D), _b2_map),
            ],
            out_specs=pl.BlockSpec((RB, D), lambda i, f, be, tot: (i, 0)),
        ),
        out_shape=jax.ShapeDtypeStruct((NROWS, D), jnp.float32),
        compiler_params=pltpu.CompilerParams(
            dimension_semantics=("arbitrary", "arbitrary"),
        ),
    )(be.reshape(128)[:NB], tot.reshape(1), xsorted, W1,
      b1.reshape(E, 1, FF), W2, b2.reshape(E, 1, D))

    ra, rb = sc_gather(ys, posa1, posb1)

    TB = 512
    out = pl.pallas_call(
        _combine_kernel,
        grid=(T // TB,),
        in_specs=[
            pl.BlockSpec((TB, D), lambda t: (t, 0)),
            pl.BlockSpec((TB, D), lambda t: (t, 0)),
            pl.BlockSpec((TB, K), lambda t: (t, 0)),
        ],
        out_specs=pl.BlockSpec((TB, D), lambda t: (t, 0)),
        out_shape=jax.ShapeDtypeStruct((T, D), jnp.float32),
    )(ra, rb, tw)

    combined = out.reshape(B, T, D)
    aux_loss = jnp.zeros((), dtype=x.dtype)
    return combined, aux_loss, ti.reshape(B, T, K), tw.reshape(B, T, K)


# speedup vs baseline: 1.6795x; 1.0875x over previous
import functools

import jax
import jax.numpy as jnp
from jax import lax
from jax.experimental import pallas as pl
from jax.experimental.pallas import tpu as pltpu
from jax.experimental.pallas import tpu_sc as plsc

B, T, D = 1, 2048, 768
FF = 3072
E = 8
K = 2

RB = 512
NB = (T * K) // RB + E - 1
NROWS = NB * RB
FFB = 3072
NF = FF // FFB

NC, NS = 2, 16
NW = NC * NS
BPW = T // NW


def _plan_kernel(x_ref, wg_ref, bg_ref,
                 tw_ref, ti_ref, posa_ref, posb_ref, be_ref, tot_ref):
    logits = jnp.dot(x_ref[...], wg_ref[...],
                     preferred_element_type=jnp.float32) + bg_ref[...]
    m = jnp.max(logits, axis=-1, keepdims=True)
    ex = jnp.exp(logits - m)
    probs = ex / jnp.sum(ex, axis=-1, keepdims=True)
    p1 = jnp.max(probs, axis=-1, keepdims=True)
    i1 = jnp.argmax(probs, axis=-1, keepdims=True)
    lane = jax.lax.broadcasted_iota(jnp.int32, probs.shape, 1)
    masked = jnp.where(lane == i1, -jnp.inf, probs)
    p2 = jnp.max(masked, axis=-1, keepdims=True)
    i2 = jnp.argmax(masked, axis=-1, keepdims=True)
    s = p1 + p2
    tw_ref[...] = jnp.concatenate([p1 / s, p2 / s], axis=1)
    ti_ref[...] = jnp.concatenate([i1, i2], axis=1).astype(jnp.int32)

    a1 = lane == i1
    a2 = lane == i2
    a = (a1 | a2).astype(jnp.int32)
    CH = 256
    af = a.astype(jnp.float32)
    tri = (jax.lax.broadcasted_iota(jnp.int32, (CH, CH), 0) >
           jax.lax.broadcasted_iota(jnp.int32, (CH, CH), 1)
           ).astype(jnp.float32)
    carry = jnp.zeros((1, E), jnp.float32)
    excl_parts = []
    for c in range(T // CH):
        ac = af[c * CH:(c + 1) * CH, :]
        excl_parts.append(
            jnp.dot(tri, ac, preferred_element_type=jnp.float32) + carry)
        carry = carry + jnp.sum(ac, axis=0, keepdims=True)
    excl = jnp.concatenate(excl_parts, axis=0).astype(jnp.int32)
    counts = carry.astype(jnp.int32)
    pc = ((counts + RB - 1) // RB) * RB
    parts = [jnp.zeros((1, 1), jnp.int32)]
    run = pc[:, 0:1]
    for e in range(1, E):
        parts.append(run)
        run = run + pc[:, e:e + 1]
    gs = jnp.concatenate(parts, axis=1)
    tot_ref[...] = run

    gsb = jnp.broadcast_to(gs, (T, E))
    posa = jnp.sum(jnp.where(a1, gsb + excl, 0), axis=1, keepdims=True)
    posb = jnp.sum(jnp.where(a2, gsb + excl, 0), axis=1, keepdims=True)
    posa_ref[...] = posa
    posb_ref[...] = posb

    lane128 = jax.lax.broadcasted_iota(jnp.int32, (1, 128), 1)
    blkstart = lane128 * RB
    acc = jnp.zeros((1, 128), jnp.int32)
    for e in range(E):
        acc += jnp.where(blkstart >= gs[:, e:e + 1], 1, 0)
    be_ref[...] = jnp.clip(acc - 1, 0, E - 1)


def _gmm_kernel(be_ref, tot_ref, xs_ref, w1_ref, b1_ref, w2_ref, b2_ref,
                ys_ref):
    i = pl.program_id(0)
    f = pl.program_id(1)

    @pl.when(i * RB < tot_ref[0])
    def _():
        h = jnp.dot(xs_ref[...], w1_ref[0],
                    preferred_element_type=jnp.float32) + b1_ref[0]
        h = jax.nn.gelu(h)
        p = jnp.dot(h, w2_ref[0], preferred_element_type=jnp.float32)

        @pl.when(f == 0)
        def _():
            ys_ref[...] = jnp.broadcast_to(b2_ref[0], ys_ref.shape)

        ys_ref[...] += p


def _last(tot_ref):
    return tot_ref[0] // RB - 1


def _w1_map(i, f, be_ref, tot_ref):
    ie = jnp.minimum(i, _last(tot_ref))
    fe = jnp.where(i <= _last(tot_ref), f, NF - 1)
    return (be_ref[ie], 0, fe)


def _b1_map(i, f, be_ref, tot_ref):
    ie = jnp.minimum(i, _last(tot_ref))
    fe = jnp.where(i <= _last(tot_ref), f, NF - 1)
    return (be_ref[ie], 0, fe)


def _w2_map(i, f, be_ref, tot_ref):
    ie = jnp.minimum(i, _last(tot_ref))
    fe = jnp.where(i <= _last(tot_ref), f, NF - 1)
    return (be_ref[ie], fe, 0)


def _b2_map(i, f, be_ref, tot_ref):
    ie = jnp.minimum(i, _last(tot_ref))
    return (be_ref[ie], 0, 0)


def _combine_kernel(ra_ref, rb_ref, tw_ref, out_ref):
    tw = tw_ref[...]
    out_ref[...] = ra_ref[...] * tw[:, 0:1] + rb_ref[...] * tw[:, 1:2]


def _sc_scatter_body(x_hbm, posa_hbm, posb_hbm, xs_hbm, idx_v, rows_v, sem):
    wid = lax.axis_index("s") * NC + lax.axis_index("c")
    base = wid * BPW
    pltpu.sync_copy(x_hbm.at[pl.ds(base, BPW)], rows_v)
    pltpu.sync_copy(posa_hbm.at[pl.ds(base, BPW)], idx_v)
    pltpu.async_copy(rows_v, xs_hbm.at[idx_v], sem).wait()
    pltpu.sync_copy(posb_hbm.at[pl.ds(base, BPW)], idx_v)
    pltpu.async_copy(rows_v, xs_hbm.at[idx_v], sem).wait()


def _sc_gather_body(ys_hbm, posa_hbm, posb_hbm, ra_hbm, rb_hbm, idx_v,
                    rows_v, sem):
    wid = lax.axis_index("s") * NC + lax.axis_index("c")
    base = wid * BPW
    pltpu.sync_copy(posa_hbm.at[pl.ds(base, BPW)], idx_v)
    pltpu.async_copy(ys_hbm.at[idx_v], rows_v, sem).wait()
    pltpu.sync_copy(rows_v, ra_hbm.at[pl.ds(base, BPW)])
    pltpu.sync_copy(posb_hbm.at[pl.ds(base, BPW)], idx_v)
    pltpu.async_copy(ys_hbm.at[idx_v], rows_v, sem).wait()
    pltpu.sync_copy(rows_v, rb_hbm.at[pl.ds(base, BPW)])


@functools.cache
def _get_sc_kernels():
    mesh = plsc.VectorSubcoreMesh(core_axis_name="c", subcore_axis_name="s",
                                  num_cores=NC, num_subcores=NS)
    scratch = [
        pltpu.VMEM((BPW,), jnp.int32),
        pltpu.VMEM((BPW, D), jnp.float32),
        pltpu.SemaphoreType.DMA,
    ]
    scatter = pl.kernel(
        _sc_scatter_body,
        out_type=jax.ShapeDtypeStruct((NROWS, D), jnp.float32),
        mesh=mesh, scratch_types=scratch)
    gather = pl.kernel(
        _sc_gather_body,
        out_type=(jax.ShapeDtypeStruct((T, D), jnp.float32),
                  jax.ShapeDtypeStruct((T, D), jnp.float32)),
        mesh=mesh, scratch_types=scratch)
    return scatter, gather


def kernel(x, Wg, bg, W1, b1, W2, b2):
    xs = x.reshape(T, D)

    tw, ti, posa, posb, be, tot = pl.pallas_call(
        _plan_kernel,
        out_shape=(
            jax.ShapeDtypeStruct((T, K), jnp.float32),
            jax.ShapeDtypeStruct((T, K), jnp.int32),
            jax.ShapeDtypeStruct((T, 1), jnp.int32),
            jax.ShapeDtypeStruct((T, 1), jnp.int32),
            jax.ShapeDtypeStruct((1, 128), jnp.int32),
            jax.ShapeDtypeStruct((1, 1), jnp.int32),
        ),
    )(xs, Wg, bg.reshape(1, E))

    posa1 = posa.reshape(T)
    posb1 = posb.reshape(T)

    sc_scatter, sc_gather = _get_sc_kernels()
    xsorted = sc_scatter(xs, posa1, posb1)

    ys = pl.pallas_call(
        _gmm_kernel,
        grid_spec=pltpu.PrefetchScalarGridSpec(
            num_scalar_prefetch=2,
            grid=(NB, NF),
            in_specs=[
                pl.BlockSpec((RB, D), lambda i, f, be, tot: (i, 0)),
                pl.BlockSpec((1, D, FFB), _w1_map),
                pl.BlockSpec((1, 1, FFB), _b1_map),
                pl.BlockSpec((1, FFB, D), _w2_map),
                pl.BlockSpec((1, 1, D), _b2_map),
            ],
            out_specs=pl.BlockSpec((RB, D), lambda i, f, be, tot: (i, 0)),
        ),
        out_shape=jax.ShapeDtypeStruct((NROWS, D), jnp.float32),
        compiler_params=pltpu.CompilerParams(
            dimension_semantics=("arbitrary", "arbitrary"),
        ),
    )(be.reshape(128)[:NB], tot.reshape(1), xsorted, W1,
      b1.reshape(E, 1, FF), W2, b2.reshape(E, 1, D))

    ra, rb = sc_gather(ys, posa1, posb1)

    TB = 512
    out = pl.pallas_call(
        _combine_kernel,
        grid=(T // TB,),
        in_specs=[
            pl.BlockSpec((TB, D), lambda t: (t, 0)),
            pl.BlockSpec((TB, D), lambda t: (t, 0)),
            pl.BlockSpec((TB, K), lambda t: (t, 0)),
        ],
        out_specs=pl.BlockSpec((TB, D), lambda t: (t, 0)),
        out_shape=jax.ShapeDtypeStruct((T, D), jnp.float32),
    )(ra, rb, tw)

    combined = out.reshape(B, T, D)
    aux_loss = jnp.zeros((), dtype=x.dtype)
    return combined, aux_loss, ti.reshape(B, T, K), tw.reshape(B, T, K)


# speedup vs baseline: 1.7763x; 1.0577x over previous
import functools

import jax
import jax.numpy as jnp
from jax import lax
from jax.experimental import pallas as pl
from jax.experimental.pallas import tpu as pltpu
from jax.experimental.pallas import tpu_sc as plsc

B, T, D = 1, 2048, 768
FF = 3072
E = 8
K = 2

RB = 640
NB = -(-(T * K) // RB) + E - 1
NROWS = NB * RB
HF = FF // 2

NC, NS = 2, 16
NW = NC * NS
BPW = T // NW


def _plan_kernel(x_ref, wg_ref, bg_ref,
                 tw_ref, ti_ref, posa_ref, posb_ref, be_ref, tot_ref):
    logits = jnp.dot(x_ref[...], wg_ref[...],
                     preferred_element_type=jnp.float32) + bg_ref[...]
    m = jnp.max(logits, axis=-1, keepdims=True)
    ex = jnp.exp(logits - m)
    probs = ex / jnp.sum(ex, axis=-1, keepdims=True)
    p1 = jnp.max(probs, axis=-1, keepdims=True)
    i1 = jnp.argmax(probs, axis=-1, keepdims=True)
    lane = jax.lax.broadcasted_iota(jnp.int32, probs.shape, 1)
    masked = jnp.where(lane == i1, -jnp.inf, probs)
    p2 = jnp.max(masked, axis=-1, keepdims=True)
    i2 = jnp.argmax(masked, axis=-1, keepdims=True)
    s = p1 + p2
    tw_ref[...] = jnp.concatenate([p1 / s, p2 / s], axis=1)
    ti_ref[...] = jnp.concatenate([i1, i2], axis=1).astype(jnp.int32)

    a1 = lane == i1
    a2 = lane == i2
    a = (a1 | a2).astype(jnp.int32)
    CH = 256
    af = a.astype(jnp.float32)
    tri = (jax.lax.broadcasted_iota(jnp.int32, (CH, CH), 0) >
           jax.lax.broadcasted_iota(jnp.int32, (CH, CH), 1)
           ).astype(jnp.float32)
    carry = jnp.zeros((1, E), jnp.float32)
    excl_parts = []
    for c in range(T // CH):
        ac = af[c * CH:(c + 1) * CH, :]
        excl_parts.append(
            jnp.dot(tri, ac, preferred_element_type=jnp.float32) + carry)
        carry = carry + jnp.sum(ac, axis=0, keepdims=True)
    excl = jnp.concatenate(excl_parts, axis=0).astype(jnp.int32)
    counts = carry.astype(jnp.int32)
    pc = ((counts + RB - 1) // RB) * RB
    parts = [jnp.zeros((1, 1), jnp.int32)]
    run = pc[:, 0:1]
    for e in range(1, E):
        parts.append(run)
        run = run + pc[:, e:e + 1]
    gs = jnp.concatenate(parts, axis=1)
    tot_ref[...] = run

    gsb = jnp.broadcast_to(gs, (T, E))
    posa = jnp.sum(jnp.where(a1, gsb + excl, 0), axis=1, keepdims=True)
    posb = jnp.sum(jnp.where(a2, gsb + excl, 0), axis=1, keepdims=True)
    posa_ref[...] = posa
    posb_ref[...] = posb

    lane128 = jax.lax.broadcasted_iota(jnp.int32, (1, 128), 1)
    blkstart = lane128 * RB
    acc = jnp.zeros((1, 128), jnp.int32)
    for e in range(E):
        acc += jnp.where(blkstart >= gs[:, e:e + 1], 1, 0)
    be_ref[...] = jnp.clip(acc - 1, 0, E - 1)


def _gmm_kernel(be_ref, tot_ref, xs_ref, w1a_ref, w1b_ref, b1_ref,
                w2a_ref, w2b_ref, b2_ref, ys_ref):
    i = pl.program_id(0)

    @pl.when(i * RB < tot_ref[0])
    def _():
        x = xs_ref[...]
        h1 = jnp.dot(x, w1a_ref[0],
                     preferred_element_type=jnp.float32) + b1_ref[0, :, :HF]
        h2 = jnp.dot(x, w1b_ref[0],
                     preferred_element_type=jnp.float32) + b1_ref[0, :, HF:]
        p = jnp.dot(jax.nn.gelu(h1), w2a_ref[0],
                    preferred_element_type=jnp.float32)
        p += jnp.dot(jax.nn.gelu(h2), w2b_ref[0],
                     preferred_element_type=jnp.float32)
        ys_ref[...] = p + b2_ref[0]


def _we_map(i, be_ref, tot_ref):
    return be_ref[jnp.minimum(i, tot_ref[0] // RB - 1)]


def _combine_kernel(ra_ref, rb_ref, tw_ref, out_ref):
    tw = tw_ref[...]
    out_ref[...] = ra_ref[...] * tw[:, 0:1] + rb_ref[...] * tw[:, 1:2]


def _sc_scatter_body(x_hbm, posa_hbm, posb_hbm, xs_hbm, idxa_v, idxb_v,
                     rows_v, sema, semb):
    wid = lax.axis_index("s") * NC + lax.axis_index("c")
    base = wid * BPW
    pltpu.sync_copy(x_hbm.at[pl.ds(base, BPW)], rows_v)
    pltpu.sync_copy(posa_hbm.at[pl.ds(base, BPW)], idxa_v)
    pltpu.sync_copy(posb_hbm.at[pl.ds(base, BPW)], idxb_v)
    cpa = pltpu.async_copy(rows_v, xs_hbm.at[idxa_v], sema)
    cpb = pltpu.async_copy(rows_v, xs_hbm.at[idxb_v], semb)
    cpa.wait()
    cpb.wait()


def _sc_gather_body(ys_hbm, posa_hbm, posb_hbm, ra_hbm, rb_hbm, idxa_v,
                    idxb_v, rowsa_v, rowsb_v, sema, semb):
    wid = lax.axis_index("s") * NC + lax.axis_index("c")
    base = wid * BPW
    pltpu.sync_copy(posa_hbm.at[pl.ds(base, BPW)], idxa_v)
    pltpu.sync_copy(posb_hbm.at[pl.ds(base, BPW)], idxb_v)
    cpa = pltpu.async_copy(ys_hbm.at[idxa_v], rowsa_v, sema)
    cpb = pltpu.async_copy(ys_hbm.at[idxb_v], rowsb_v, semb)
    cpa.wait()
    cpb.wait()
    pltpu.sync_copy(rowsa_v, ra_hbm.at[pl.ds(base, BPW)])
    pltpu.sync_copy(rowsb_v, rb_hbm.at[pl.ds(base, BPW)])


@functools.cache
def _get_sc_kernels():
    mesh = plsc.VectorSubcoreMesh(core_axis_name="c", subcore_axis_name="s",
                                  num_cores=NC, num_subcores=NS)
    scatter = pl.kernel(
        _sc_scatter_body,
        out_type=jax.ShapeDtypeStruct((NROWS, D), jnp.float32),
        mesh=mesh, scratch_types=[
            pltpu.VMEM((BPW,), jnp.int32),
            pltpu.VMEM((BPW,), jnp.int32),
            pltpu.VMEM((BPW, D), jnp.float32),
            pltpu.SemaphoreType.DMA,
            pltpu.SemaphoreType.DMA,
        ])
    gather = pl.kernel(
        _sc_gather_body,
        out_type=(jax.ShapeDtypeStruct((T, D), jnp.float32),
                  jax.ShapeDtypeStruct((T, D), jnp.float32)),
        mesh=mesh, scratch_types=[
            pltpu.VMEM((BPW,), jnp.int32),
            pltpu.VMEM((BPW,), jnp.int32),
            pltpu.VMEM((BPW, D), jnp.float32),
            pltpu.VMEM((BPW, D), jnp.float32),
            pltpu.SemaphoreType.DMA,
            pltpu.SemaphoreType.DMA,
        ])
    return scatter, gather


def kernel(x, Wg, bg, W1, b1, W2, b2):
    xs = x.reshape(T, D)

    tw, ti, posa, posb, be, tot = pl.pallas_call(
        _plan_kernel,
        out_shape=(
            jax.ShapeDtypeStruct((T, K), jnp.float32),
            jax.ShapeDtypeStruct((T, K), jnp.int32),
            jax.ShapeDtypeStruct((T, 1), jnp.int32),
            jax.ShapeDtypeStruct((T, 1), jnp.int32),
            jax.ShapeDtypeStruct((1, 128), jnp.int32),
            jax.ShapeDtypeStruct((1, 1), jnp.int32),
        ),
    )(xs, Wg, bg.reshape(1, E))

    posa1 = posa.reshape(T)
    posb1 = posb.reshape(T)

    sc_scatter, sc_gather = _get_sc_kernels()
    xsorted = sc_scatter(xs, posa1, posb1)

    ys = pl.pallas_call(
        _gmm_kernel,
        grid_spec=pltpu.PrefetchScalarGridSpec(
            num_scalar_prefetch=2,
            grid=(NB,),
            in_specs=[
                pl.BlockSpec((RB, D), lambda i, be, tot: (i, 0)),
                pl.BlockSpec((1, D, HF),
                             lambda i, be, tot: (_we_map(i, be, tot), 0, 0)),
                pl.BlockSpec((1, D, HF),
                             lambda i, be, tot: (_we_map(i, be, tot), 0, 1)),
                pl.BlockSpec((1, 1, FF),
                             lambda i, be, tot: (_we_map(i, be, tot), 0, 0)),
                pl.BlockSpec((1, HF, D),
                             lambda i, be, tot: (_we_map(i, be, tot), 0, 0)),
                pl.BlockSpec((1, HF, D),
                             lambda i, be, tot: (_we_map(i, be, tot), 1, 0)),
                pl.BlockSpec((1, 1, D),
                             lambda i, be, tot: (_we_map(i, be, tot), 0, 0)),
            ],
            out_specs=pl.BlockSpec((RB, D), lambda i, be, tot: (i, 0)),
        ),
        out_shape=jax.ShapeDtypeStruct((NROWS, D), jnp.float32),
        compiler_params=pltpu.CompilerParams(
            dimension_semantics=("arbitrary",),
        ),
    )(be.reshape(128)[:NB], tot.reshape(1), xsorted, W1, W1,
      b1.reshape(E, 1, FF), W2, W2, b2.reshape(E, 1, D))

    ra, rb = sc_gather(ys, posa1, posb1)

    TB = 512
    out = pl.pallas_call(
        _combine_kernel,
        grid=(T // TB,),
        in_specs=[
            pl.BlockSpec((TB, D), lambda t: (t, 0)),
            pl.BlockSpec((TB, D), lambda t: (t, 0)),
            pl.BlockSpec((TB, K), lambda t: (t, 0)),
        ],
        out_specs=pl.BlockSpec((TB, D), lambda t: (t, 0)),
        out_shape=jax.ShapeDtypeStruct((T, D), jnp.float32),
    )(ra, rb, tw)

    combined = out.reshape(B, T, D)
    aux_loss = jnp.zeros((), dtype=x.dtype)
    return combined, aux_loss, ti.reshape(B, T, K), tw.reshape(B, T, K)
